# baseline (speedup 1.0000x reference)
import functools

import jax
import jax.numpy as jnp
from jax import lax
from jax.experimental import pallas as pl
from jax.experimental.pallas import tpu as pltpu
from jax.experimental.pallas import tpu_sc as plsc

B = 1024
L = 200
D = 128
MAXLEN = 200
NC = 2
NS = 16
NW = NC * NS
ROWS_PER_W = B // NW
TOK_PER_W = ROWS_PER_W * L
TPAD = 208
CH = 128
NCHUNK = TOK_PER_W // CH


def _table_body(pos_ref, mod_ref, out_ref):
    pos = pos_ref[...]
    mod = mod_ref[...]
    pad = jnp.broadcast_to(jnp.zeros_like(mod), (TPAD - MAXLEN, D))
    out_ref[...] = jnp.concatenate([pos, pad], axis=0) + mod


def _build_table(pos200, mod2d):
    return pl.pallas_call(
        _table_body,
        out_shape=jax.ShapeDtypeStruct((TPAD, D), jnp.float32),
    )(pos200, mod2d)


def _sc_body(ids_hbm, mask_hbm, tok_tab, pos_tab, x_hbm, xe_hbm,
             ids_v, mask_v, pid_v, tbuf0, tbuf1, pbuf0, pbuf1,
             sgt0, sgp0, swt0, swp0, sgt1, sgp1, swt1, swp1):
    wid = lax.axis_index("s") * NC + lax.axis_index("c")
    tok0 = wid * TOK_PER_W

    pltpu.sync_copy(ids_hbm.at[pl.ds(tok0, TOK_PER_W)],
                    ids_v.at[pl.ds(0, TOK_PER_W)])
    pltpu.sync_copy(mask_hbm.at[pl.ds(tok0, TOK_PER_W)],
                    mask_v.at[pl.ds(0, TOK_PER_W)])

    lane = lax.iota(jnp.int32, 16)

    def pid_row(r, _):
        rb = r * L
        carry = jnp.int32(-1)
        for i in range(12):
            off = rb + i * 16
            m = mask_v[pl.ds(off, 16)]
            nm = jnp.where(m != 0, 0, 1)
            pid = plsc.cumsum(nm) + carry
            pid_v[pl.ds(off, 16)] = jnp.where(m != 0, MAXLEN, pid)
            carry = carry + jnp.sum(nm)
        off = rb + 192
        m = mask_v[pl.ds(off, 16)]
        nm = jnp.where(jnp.logical_and(lane < 8, m == 0), 1, 0)
        pid = plsc.cumsum(nm) + carry
        pid_v[pl.ds(off, 16)] = jnp.where(m != 0, MAXLEN, pid)
        return 0

    lax.fori_loop(0, ROWS_PER_W, pid_row, 0)

    slots = ((tbuf0, pbuf0, sgt0, sgp0, swt0, swp0),
             (tbuf1, pbuf1, sgt1, sgp1, swt1, swp1))

    def issue_gather(c, s):
        tb, pb, sgt, sgp, _, _ = slots[s]
        off = c * CH
        pltpu.async_copy(tok_tab.at[ids_v.at[pl.ds(off, CH)]], tb, sgt)
        pltpu.async_copy(pos_tab.at[pid_v.at[pl.ds(off, CH)]], pb, sgp)

    def wait_gather(s):
        tb, pb, sgt, sgp, _, _ = slots[s]
        pltpu.make_async_copy(tok_tab.at[pl.ds(0, CH)], tb, sgt).wait()
        pltpu.make_async_copy(tok_tab.at[pl.ds(0, CH)], pb, sgp).wait()

    def issue_writeout(c, s):
        tb, pb, _, _, swt, swp = slots[s]
        off = tok0 + c * CH
        pltpu.async_copy(tb, x_hbm.at[pl.ds(off, CH)], swt)
        pltpu.async_copy(pb, xe_hbm.at[pl.ds(off, CH)], swp)

    def wait_writeout(s):
        tb, pb, _, _, swt, swp = slots[s]
        pltpu.make_async_copy(tb, x_hbm.at[pl.ds(0, CH)], swt).wait()
        pltpu.make_async_copy(pb, xe_hbm.at[pl.ds(0, CH)], swp).wait()

    issue_gather(0, 0)
    issue_gather(1, 1)

    @pl.loop(0, NCHUNK // 2 - 1)
    def _pipe(i):
        c = i * 2
        wait_gather(0)
        issue_writeout(c, 0)
        wait_gather(1)
        issue_writeout(c + 1, 1)
        wait_writeout(0)
        issue_gather(c + 2, 0)
        wait_writeout(1)
        issue_gather(c + 3, 1)

    wait_gather(0)
    issue_writeout(NCHUNK - 2, 0)
    wait_gather(1)
    issue_writeout(NCHUNK - 1, 1)
    wait_writeout(0)
    wait_writeout(1)


_sc_gather = pl.kernel(
    _sc_body,
    out_type=(jax.ShapeDtypeStruct((B * L, D), jnp.float32),
              jax.ShapeDtypeStruct((B * L, D), jnp.float32)),
    mesh=plsc.VectorSubcoreMesh(core_axis_name="c", subcore_axis_name="s"),
    compiler_params=pltpu.CompilerParams(needs_layout_passes=False),
    scratch_types=[
        pltpu.VMEM((TOK_PER_W + 16,), jnp.int32),
        pltpu.VMEM((TOK_PER_W + 16,), jnp.int32),
        pltpu.VMEM((TOK_PER_W + 16,), jnp.int32),
        pltpu.VMEM((CH, D), jnp.float32),
        pltpu.VMEM((CH, D), jnp.float32),
        pltpu.VMEM((CH, D), jnp.float32),
        pltpu.VMEM((CH, D), jnp.float32),
    ] + [pltpu.SemaphoreType.DMA] * 8,
)


def kernel(tensor, target_mask, token_emb, mod_emb, pos_emb):
    ids = tensor.reshape(B * L)
    mask = target_mask.astype(jnp.int32).reshape(B * L)
    table = _build_table(pos_emb[0, :MAXLEN, :], mod_emb[0])
    x_flat, xe_flat = _sc_gather(ids, mask, token_emb, table)
    return (x_flat.reshape(B, L, D), xe_flat.reshape(B, L, D), tensor)
